# baseline (speedup 1.0000x reference)
import dataclasses
import functools

import jax
import jax.numpy as jnp
from jax import lax
from jax.experimental import pallas as pl
from jax.experimental.pallas import tpu as pltpu
from jax.experimental.pallas import tpu_sc as plsc

_N = 1000012
_NP = 1000064
_B = 16384
_F = 26
_K = 16
_NIDX = _B * _F
_NC, _NS = 2, 16
_NW = _NC * _NS
_PER_W = _NIDX // _NW
_CH = 1664
_RCH = _CH // _F
_NSTEP = _PER_W // _CH

_TC = 3968
_TB = 992
_NT = 999936 // _TC
_TSL = 256
_TPW = _TSL // _NW

_R = 1024


def _sc_transpose(emb_t, tailp):
    mesh = plsc.VectorSubcoreMesh(core_axis_name="c", subcore_axis_name="s")
    cp = pltpu.CompilerParams()
    if "needs_layout_passes" in pltpu.CompilerParams.__dataclass_fields__:
        cp = dataclasses.replace(cp, needs_layout_passes=False)

    @functools.partial(
        pl.kernel,
        mesh=mesh,
        compiler_params=cp,
        out_type=jax.ShapeDtypeStruct((_NP * _K,), jnp.float32),
        scratch_types=[
            pltpu.VMEM((_K, _TC), jnp.float32),
            pltpu.VMEM((_TB * _K,), jnp.float32),
            pltpu.VMEM((2048,), jnp.float32),
        ],
    )
    def k(et_hbm, tl_hbm, lin_hbm, vbuf, obuf, rbuf):
        wid = lax.axis_index("s") * _NC + lax.axis_index("c")
        iota16 = lax.iota(jnp.int32, 16)
        for i in range(_TPW):
            t = wid * _TPW + i

            @pl.when(t < _NT)
            def _():
                c0 = t * _TC
                pltpu.sync_copy(et_hbm.at[pl.ds(0, _K), pl.ds(c0, _TC)], vbuf)
                for bb in range(_TC // _TB):

                    @pl.loop(0, _TB)
                    def _(j):
                        col = jnp.zeros((16,), jnp.int32) + (bb * _TB + j)
                        obuf[pl.ds(j * _K, 16)] = plsc.load_gather(
                            vbuf, [iota16, col])

                    pltpu.sync_copy(
                        obuf,
                        lin_hbm.at[pl.ds((c0 + bb * _TB) * _K, _TB * _K)])

            @pl.when(t == _NT)
            def _():
                pltpu.sync_copy(tl_hbm, rbuf)
                pltpu.sync_copy(rbuf, lin_hbm.at[pl.ds(999936 * _K, 2048)])

    return k(emb_t, tailp)


def _sc_fm(xf, emb_rm, fc1):
    mesh = plsc.VectorSubcoreMesh(core_axis_name="c", subcore_axis_name="s")

    @functools.partial(
        pl.kernel,
        mesh=mesh,
        compiler_params=pltpu.CompilerParams(use_tc_tiling_on_sc=False),
        out_type=(
            jax.ShapeDtypeStruct((_B, _K), jnp.float32),
            jax.ShapeDtypeStruct((_NIDX,), jnp.float32),
        ),
        scratch_types=[
            pltpu.VMEM((_CH,), jnp.int32),
            pltpu.VMEM((_CH, _K), jnp.float32),
            pltpu.VMEM((_CH,), jnp.float32),
            pltpu.VMEM((_RCH, _K), jnp.float32),
            pltpu.SemaphoreType.DMA,
        ],
    )
    def k(x_hbm, emb_hbm, fc_hbm, t_out, f_out, idxb, ebuf, fbuf, tbuf, sem):
        wid = lax.axis_index("s") * _NC + lax.axis_index("c")
        base = wid * _PER_W
        rbase = wid * (_PER_W // _F)
        for step in range(_NSTEP):
            j0 = base + step * _CH
            r0 = rbase + step * _RCH
            pltpu.sync_copy(x_hbm.at[pl.ds(j0, _CH)], idxb)
            cp1 = pltpu.async_copy(emb_hbm.at[idxb], ebuf, sem)
            cp2 = pltpu.async_copy(fc_hbm.at[idxb], fbuf, sem)
            cp1.wait()
            cp2.wait()
            pltpu.sync_copy(fbuf, f_out.at[pl.ds(j0, _CH)])

            @pl.loop(0, _RCH)
            def _(r):
                p = r * _F
                s = ebuf[p, :]
                ss = s * s
                for f in range(1, _F):
                    v = ebuf[p + f, :]
                    s = s + v
                    ss = ss + v * v
                tbuf[r, :] = s * s - ss

            pltpu.sync_copy(tbuf, t_out.at[pl.ds(r0, _RCH)])

    return k(xf, emb_rm, fc1)


def _fin_body(t_ref, fc_ref, w_ref, b_ref, o_ref):
    inter = 0.5 * jnp.sum(t_ref[...], axis=1)
    fcs = jnp.sum(fc_ref[...], axis=1)
    z = fcs * w_ref[0, 0] + b_ref[0] + inter
    o_ref[...] = jax.nn.sigmoid(z)


def _tc_finish(t2, fc2, W, b):
    return pl.pallas_call(
        _fin_body,
        grid=(_B // _R,),
        in_specs=[
            pl.BlockSpec((_R, _K), lambda i: (i, 0)),
            pl.BlockSpec((_R, _F), lambda i: (i, 0)),
            pl.BlockSpec(memory_space=pltpu.SMEM),
            pl.BlockSpec(memory_space=pltpu.SMEM),
        ],
        out_specs=pl.BlockSpec((_R,), lambda i: (i,)),
        out_shape=jax.ShapeDtypeStruct((_B,), jnp.float32),
        compiler_params=pltpu.CompilerParams(
            dimension_semantics=("parallel",)),
    )(t2, fc2, W, b)


def kernel(x, emb_table, fc_table, W, b):
    tail = emb_table[999936:, :]
    tailp = jnp.pad(tail, ((0, 52), (0, 0))).reshape(2048)
    emb_lin = _sc_transpose(emb_table.T, tailp)
    emb_rm = emb_lin.reshape(_NP, _K)
    fc1 = fc_table.reshape(_N)
    xf = x.reshape(_NIDX)
    t2, fcv = _sc_fm(xf, emb_rm, fc1)
    return _tc_finish(t2, fcv.reshape(_B, _F), W, b)


# speedup vs baseline: 1.3794x; 1.3794x over previous
import dataclasses
import functools

import jax
import jax.numpy as jnp
from jax import lax
from jax.experimental import pallas as pl
from jax.experimental.pallas import tpu as pltpu
from jax.experimental.pallas import tpu_sc as plsc

_N = 1000012
_NP = 1000064
_B = 16384
_F = 26
_K = 16
_NIDX = _B * _F
_NC, _NS = 2, 16
_NW = _NC * _NS
_PER_W = _NIDX // _NW
_CH = 1664
_RCH = _CH // _F
_NSTEP = _PER_W // _CH

_TC = 3968
_TB = 992
_NT = 999936 // _TC
_TSL = 256
_TPW = _TSL // _NW

_R = 1024


def _sc_transpose(emb_t, tailp):
    mesh = plsc.VectorSubcoreMesh(core_axis_name="c", subcore_axis_name="s")
    cp = pltpu.CompilerParams()
    if "needs_layout_passes" in pltpu.CompilerParams.__dataclass_fields__:
        cp = dataclasses.replace(cp, needs_layout_passes=False)

    @functools.partial(
        pl.kernel,
        mesh=mesh,
        compiler_params=cp,
        out_type=jax.ShapeDtypeStruct((_NP * _K,), jnp.float32),
        scratch_types=[
            pltpu.VMEM((_K, _TC), jnp.float32),
            pltpu.VMEM((_TB * _K,), jnp.float32),
            pltpu.VMEM((2048,), jnp.float32),
        ],
    )
    def k(et_hbm, tl_hbm, lin_hbm, vbuf, obuf, rbuf):
        wid = lax.axis_index("s") * _NC + lax.axis_index("c")
        iota16 = lax.iota(jnp.int32, 16)
        for i in range(_TPW):
            t = wid * _TPW + i

            @pl.when(t < _NT)
            def _():
                c0 = t * _TC
                pltpu.sync_copy(et_hbm.at[pl.ds(0, _K), pl.ds(c0, _TC)], vbuf)
                for bb in range(_TC // _TB):

                    @plsc.parallel_loop(0, _TB, unroll=8)
                    def _(j):
                        col = jnp.zeros((16,), jnp.int32) + (bb * _TB + j)
                        obuf[pl.ds(j * _K, 16)] = plsc.load_gather(
                            vbuf, [iota16, col])

                    pltpu.sync_copy(
                        obuf,
                        lin_hbm.at[pl.ds((c0 + bb * _TB) * _K, _TB * _K)])

            @pl.when(t == _NT)
            def _():
                pltpu.sync_copy(tl_hbm, rbuf)
                pltpu.sync_copy(rbuf, lin_hbm.at[pl.ds(999936 * _K, 2048)])

    return k(emb_t, tailp)


def _sc_fm(xf, emb_rm, fc1):
    mesh = plsc.VectorSubcoreMesh(core_axis_name="c", subcore_axis_name="s")

    @functools.partial(
        pl.kernel,
        mesh=mesh,
        compiler_params=pltpu.CompilerParams(use_tc_tiling_on_sc=False),
        out_type=(
            jax.ShapeDtypeStruct((_B, _K), jnp.float32),
            jax.ShapeDtypeStruct((_NIDX,), jnp.float32),
        ),
        scratch_types=[
            pltpu.VMEM((_CH,), jnp.int32),
            pltpu.VMEM((_CH, _K), jnp.float32),
            pltpu.VMEM((_CH,), jnp.float32),
            pltpu.VMEM((_RCH, _K), jnp.float32),
            pltpu.SemaphoreType.DMA,
        ],
    )
    def k(x_hbm, emb_hbm, fc_hbm, t_out, f_out, idxb, ebuf, fbuf, tbuf, sem):
        wid = lax.axis_index("s") * _NC + lax.axis_index("c")
        base = wid * _PER_W
        rbase = wid * (_PER_W // _F)
        for step in range(_NSTEP):
            j0 = base + step * _CH
            r0 = rbase + step * _RCH
            pltpu.sync_copy(x_hbm.at[pl.ds(j0, _CH)], idxb)
            cp1 = pltpu.async_copy(emb_hbm.at[idxb], ebuf, sem)
            cp2 = pltpu.async_copy(fc_hbm.at[idxb], fbuf, sem)
            cp1.wait()
            cp2.wait()
            pltpu.sync_copy(fbuf, f_out.at[pl.ds(j0, _CH)])

            @pl.loop(0, _RCH)
            def _(r):
                p = r * _F
                s = ebuf[p, :]
                ss = s * s
                for f in range(1, _F):
                    v = ebuf[p + f, :]
                    s = s + v
                    ss = ss + v * v
                tbuf[r, :] = s * s - ss

            pltpu.sync_copy(tbuf, t_out.at[pl.ds(r0, _RCH)])

    return k(xf, emb_rm, fc1)


def _fin_body(t_ref, fc_ref, w_ref, b_ref, o_ref):
    inter = 0.5 * jnp.sum(t_ref[...], axis=1)
    fcs = jnp.sum(fc_ref[...], axis=1)
    z = fcs * w_ref[0, 0] + b_ref[0] + inter
    o_ref[...] = jax.nn.sigmoid(z)


def _tc_finish(t2, fc2, W, b):
    return pl.pallas_call(
        _fin_body,
        grid=(_B // _R,),
        in_specs=[
            pl.BlockSpec((_R, _K), lambda i: (i, 0)),
            pl.BlockSpec((_R, _F), lambda i: (i, 0)),
            pl.BlockSpec(memory_space=pltpu.SMEM),
            pl.BlockSpec(memory_space=pltpu.SMEM),
        ],
        out_specs=pl.BlockSpec((_R,), lambda i: (i,)),
        out_shape=jax.ShapeDtypeStruct((_B,), jnp.float32),
        compiler_params=pltpu.CompilerParams(
            dimension_semantics=("parallel",)),
    )(t2, fc2, W, b)


def kernel(x, emb_table, fc_table, W, b):
    tail = emb_table[999936:, :]
    tailp = jnp.pad(tail, ((0, 52), (0, 0))).reshape(2048)
    emb_lin = _sc_transpose(emb_table.T, tailp)
    emb_rm = emb_lin.reshape(_NP, _K)
    fc1 = fc_table.reshape(_N)
    xf = x.reshape(_NIDX)
    t2, fcv = _sc_fm(xf, emb_rm, fc1)
    return _tc_finish(t2, fcv.reshape(_B, _F), W, b)


# speedup vs baseline: 1.3816x; 1.0016x over previous
import dataclasses
import functools

import jax
import jax.numpy as jnp
from jax import lax
from jax.experimental import pallas as pl
from jax.experimental.pallas import tpu as pltpu
from jax.experimental.pallas import tpu_sc as plsc

_N = 1000012
_NP = 1000064
_B = 16384
_F = 26
_K = 16
_NIDX = _B * _F
_NC, _NS = 2, 16
_NW = _NC * _NS
_PER_W = _NIDX // _NW
_CH = 1664
_RCH = _CH // _F
_NSTEP = _PER_W // _CH

_TC = 3968
_TB = 992
_NT = 999936 // _TC
_TSL = 256
_TPW = _TSL // _NW

_R = 1024


def _sc_transpose(emb_t, tailp):
    mesh = plsc.VectorSubcoreMesh(core_axis_name="c", subcore_axis_name="s")
    cp = pltpu.CompilerParams()
    if "needs_layout_passes" in pltpu.CompilerParams.__dataclass_fields__:
        cp = dataclasses.replace(cp, needs_layout_passes=False)

    @functools.partial(
        pl.kernel,
        mesh=mesh,
        compiler_params=cp,
        out_type=jax.ShapeDtypeStruct((_NP * _K,), jnp.float32),
        scratch_types=[
            pltpu.VMEM((_K, _TC + 4), jnp.float32),
            pltpu.VMEM((_TB * _K,), jnp.float32),
            pltpu.VMEM((2048,), jnp.float32),
        ],
    )
    def k(et_hbm, tl_hbm, lin_hbm, vbuf, obuf, rbuf):
        wid = lax.axis_index("s") * _NC + lax.axis_index("c")
        iota16 = lax.iota(jnp.int32, 16)
        for i in range(_TPW):
            t = wid * _TPW + i

            @pl.when(t < _NT)
            def _():
                c0 = t * _TC
                pltpu.sync_copy(et_hbm.at[pl.ds(0, _K), pl.ds(c0, _TC)],
                                vbuf.at[:, pl.ds(0, _TC)])
                for bb in range(_TC // _TB):

                    @plsc.parallel_loop(0, _TB, unroll=8)
                    def _(j):
                        col = jnp.zeros((16,), jnp.int32) + (bb * _TB + j)
                        obuf[pl.ds(j * _K, 16)] = plsc.load_gather(
                            vbuf, [iota16, col])

                    pltpu.sync_copy(
                        obuf,
                        lin_hbm.at[pl.ds((c0 + bb * _TB) * _K, _TB * _K)])

            @pl.when(t == _NT)
            def _():
                pltpu.sync_copy(tl_hbm, rbuf)
                pltpu.sync_copy(rbuf, lin_hbm.at[pl.ds(999936 * _K, 2048)])

    return k(emb_t, tailp)


def _sc_fm(xf, emb_rm, fc1):
    mesh = plsc.VectorSubcoreMesh(core_axis_name="c", subcore_axis_name="s")

    @functools.partial(
        pl.kernel,
        mesh=mesh,
        compiler_params=pltpu.CompilerParams(use_tc_tiling_on_sc=False),
        out_type=(
            jax.ShapeDtypeStruct((_B, _K), jnp.float32),
            jax.ShapeDtypeStruct((_NIDX,), jnp.float32),
        ),
        scratch_types=[
            pltpu.VMEM((_CH,), jnp.int32),
            pltpu.VMEM((_CH, _K), jnp.float32),
            pltpu.VMEM((_CH,), jnp.float32),
            pltpu.VMEM((_RCH, _K), jnp.float32),
            pltpu.SemaphoreType.DMA,
        ],
    )
    def k(x_hbm, emb_hbm, fc_hbm, t_out, f_out, idxb, ebuf, fbuf, tbuf, sem):
        wid = lax.axis_index("s") * _NC + lax.axis_index("c")
        base = wid * _PER_W
        rbase = wid * (_PER_W // _F)
        for step in range(_NSTEP):
            j0 = base + step * _CH
            r0 = rbase + step * _RCH
            pltpu.sync_copy(x_hbm.at[pl.ds(j0, _CH)], idxb)
            cp1 = pltpu.async_copy(emb_hbm.at[idxb], ebuf, sem)
            cp2 = pltpu.async_copy(fc_hbm.at[idxb], fbuf, sem)
            cp1.wait()
            cp2.wait()
            pltpu.sync_copy(fbuf, f_out.at[pl.ds(j0, _CH)])

            @pl.loop(0, _RCH)
            def _(r):
                p = r * _F
                s = ebuf[p, :]
                ss = s * s
                for f in range(1, _F):
                    v = ebuf[p + f, :]
                    s = s + v
                    ss = ss + v * v
                tbuf[r, :] = s * s - ss

            pltpu.sync_copy(tbuf, t_out.at[pl.ds(r0, _RCH)])

    return k(xf, emb_rm, fc1)


def _fin_body(t_ref, fc_ref, w_ref, b_ref, o_ref):
    inter = 0.5 * jnp.sum(t_ref[...], axis=1)
    fcs = jnp.sum(fc_ref[...], axis=1)
    z = fcs * w_ref[0, 0] + b_ref[0] + inter
    o_ref[...] = jax.nn.sigmoid(z)


def _tc_finish(t2, fc2, W, b):
    return pl.pallas_call(
        _fin_body,
        grid=(_B // _R,),
        in_specs=[
            pl.BlockSpec((_R, _K), lambda i: (i, 0)),
            pl.BlockSpec((_R, _F), lambda i: (i, 0)),
            pl.BlockSpec(memory_space=pltpu.SMEM),
            pl.BlockSpec(memory_space=pltpu.SMEM),
        ],
        out_specs=pl.BlockSpec((_R,), lambda i: (i,)),
        out_shape=jax.ShapeDtypeStruct((_B,), jnp.float32),
        compiler_params=pltpu.CompilerParams(
            dimension_semantics=("parallel",)),
    )(t2, fc2, W, b)


def kernel(x, emb_table, fc_table, W, b):
    tail = emb_table[999936:, :]
    tailp = jnp.pad(tail, ((0, 52), (0, 0))).reshape(2048)
    emb_lin = _sc_transpose(emb_table.T, tailp)
    emb_rm = emb_lin.reshape(_NP, _K)
    fc1 = fc_table.reshape(_N)
    xf = x.reshape(_NIDX)
    t2, fcv = _sc_fm(xf, emb_rm, fc1)
    return _tc_finish(t2, fcv.reshape(_B, _F), W, b)


# speedup vs baseline: 1.5869x; 1.1486x over previous
import functools

import jax
import jax.numpy as jnp
from jax import lax
from jax.experimental import pallas as pl
from jax.experimental.pallas import tpu as pltpu
from jax.experimental.pallas import tpu_sc as plsc

_N = 1000012
_SP = 1000064
_B = 16384
_F = 26
_K = 16
_NIDX = _B * _F
_NC, _NS = 2, 16
_NW = _NC * _NS
_RW = _B // _NW
_RCH = 64
_CH = _RCH * _F
_NSTEP = _RW // _RCH
_PER_W = _RW * _F

_CC = 55552
_NBIG = _N // _CC
_REM = _N - _NBIG * _CC
_TPP = 20
_TPW = _K * _TPP // _NW


def _sc_relayout(emb_t, tailp):
    mesh = plsc.VectorSubcoreMesh(core_axis_name="c", subcore_axis_name="s")

    @functools.partial(
        pl.kernel,
        mesh=mesh,
        out_type=jax.ShapeDtypeStruct((_K * _SP,), jnp.float32),
        scratch_types=[
            pltpu.VMEM((_CC,), jnp.float32),
            pltpu.VMEM((128,), jnp.float32),
        ],
    )
    def k(et_hbm, tl_hbm, lin_hbm, buf, rbuf):
        wid = lax.axis_index("s") * _NC + lax.axis_index("c")
        for i in range(_TPW):
            t = wid * _TPW + i
            kk = t // _TPP
            sub = t % _TPP

            @pl.when(sub < _NBIG)
            def _():
                off = sub * _CC
                pltpu.sync_copy(et_hbm.at[kk].at[pl.ds(off, _CC)], buf)
                pltpu.sync_copy(buf, lin_hbm.at[pl.ds(kk * _SP + off, _CC)])

            @pl.when(sub == _NBIG)
            def _():
                off = _NBIG * _CC
                pltpu.sync_copy(tl_hbm.at[pl.ds(kk * 128, 128)], rbuf)
                pltpu.sync_copy(rbuf, lin_hbm.at[pl.ds(kk * _SP + off, 128)])

    return k(emb_t, tailp)


def _sc_fm(xp, et1, fc1, W, b):
    mesh = plsc.VectorSubcoreMesh(core_axis_name="c", subcore_axis_name="s")

    @functools.partial(
        pl.kernel,
        mesh=mesh,
        compiler_params=pltpu.CompilerParams(use_tc_tiling_on_sc=False),
        out_type=jax.ShapeDtypeStruct((_B,), jnp.float32),
        scratch_types=[
            pltpu.VMEM((_CH,), jnp.int32),
            pltpu.VMEM((_CH,), jnp.int32),
            pltpu.VMEM((_K, _CH), jnp.float32),
            pltpu.VMEM((_K, _CH), jnp.float32),
            pltpu.VMEM((_CH,), jnp.float32),
            pltpu.VMEM((_CH,), jnp.float32),
            pltpu.VMEM((_RCH,), jnp.float32),
            pltpu.VMEM((16,), jnp.float32),
            pltpu.VMEM((16,), jnp.float32),
            pltpu.SemaphoreType.DMA,
            pltpu.SemaphoreType.DMA,
        ],
    )
    def k(x_hbm, et_hbm, fc_hbm, w_hbm, b_hbm, o_hbm,
          idxb0, idxb1, ebuf0, ebuf1, fbuf0, fbuf1, obuf, wvm, bvm,
          sem0, sem1):
        pltpu.sync_copy(w_hbm, wvm)
        pltpu.sync_copy(b_hbm, bvm)
        w0 = wvm[...]
        b0 = bvm[...]
        wid = lax.axis_index("s") * _NC + lax.axis_index("c")
        base = wid * _PER_W
        rbase = wid * _RW
        sets = [(idxb0, ebuf0, fbuf0, sem0), (idxb1, ebuf1, fbuf1, sem1)]

        def fire(step, st):
            idxb, ebuf, fbuf, sem = st
            j0 = base + step * _CH
            pltpu.sync_copy(x_hbm.at[pl.ds(j0, _CH)], idxb)
            cps = []
            for kk in range(_K):
                src = et_hbm.at[pl.ds(kk * _SP, _N)]
                cps.append(pltpu.async_copy(src.at[idxb], ebuf.at[kk], sem))
            cps.append(pltpu.async_copy(fc_hbm.at[idxb], fbuf, sem))
            return cps

        cps = fire(0, sets[0])
        for step in range(_NSTEP):
            _, ebuf, fbuf, _ = sets[step % 2]
            cur_cps = cps
            if step + 1 < _NSTEP:
                cps = fire(step + 1, sets[(step + 1) % 2])
            for cp in cur_cps:
                cp.wait()

            @pl.loop(0, _RCH, step=16)
            def _(m):
                def kbody(kk, tacc):
                    s = ebuf[kk, pl.ds(m, 16)]
                    ss = s * s
                    for f in range(1, _F):
                        v = ebuf[kk, pl.ds(f * _RCH + m, 16)]
                        s = s + v
                        ss = ss + v * v
                    return tacc + s * s - ss

                t = lax.fori_loop(0, _K, kbody, jnp.zeros(16, jnp.float32))
                fcs = fbuf[pl.ds(m, 16)]
                for f in range(1, _F):
                    fcs = fcs + fbuf[pl.ds(f * _RCH + m, 16)]
                z = fcs * w0 + b0 + 0.5 * t
                obuf[pl.ds(m, 16)] = 1.0 / (1.0 + jnp.exp(-z))

            pltpu.sync_copy(obuf, o_hbm.at[pl.ds(rbase + step * _RCH, _RCH)])

    return k(xp, et1, fc1, W, b)


def kernel(x, emb_table, fc_table, W, b):
    tail = emb_table[_NBIG * _CC:, :]
    tailp = jnp.pad(tail, ((0, 128 - _REM), (0, 0))).T.reshape(_K * 128)
    et1 = _sc_relayout(emb_table.T, tailp)
    fc1 = fc_table.reshape(_N)
    xp = (x.reshape(_NW, _NSTEP, _RCH, _F)
          .transpose(0, 1, 3, 2)
          .reshape(_NIDX))
    w16 = jnp.broadcast_to(W.reshape(1), (16,))
    b16 = jnp.broadcast_to(b, (16,))
    return _sc_fm(xp, et1, fc1, w16, b16)
